# baseline (speedup 1.0000x reference)
import functools

import jax
import jax.numpy as jnp
from jax import lax
from jax.experimental import pallas as pl
from jax.experimental.pallas import tpu as pltpu
from jax.experimental.pallas import tpu_sc as plsc

_N = 10000
_D = 128
_G = 128
_E = 320000

_NC = 2
_NS = 16
_CHUNK = 128
_CPW = 80
_CPW0 = 80
_CPW1 = 80
_HP = 40
_NB = 2
_NCHUNK = _NC * _NS * _CPW
_EPAD = _NCHUNK * _CHUNK
_ACC_ROWS = 10112
_ZPT = _ACC_ROWS // _NS
_DUMMY = _N


def _sc_mesh():
    return plsc.VectorSubcoreMesh(
        core_axis_name="c", subcore_axis_name="s",
        num_cores=_NC, num_subcores=_NS)


def _zero_acc(zb_v, acc, s):
    for i in range(16):
        for j in range(_D // 16):
            zb_v[i, pl.ds(j * 16, 16)] = jnp.zeros((16,), jnp.float32)

    def zero_body(k, _):
        pltpu.sync_copy(zb_v, acc.at[pl.ds(s * _ZPT + k * 16, 16), :])
        return 0

    lax.fori_loop(0, _ZPT // 16, zero_body, 0)
    if _ZPT % 16:
        pltpu.sync_copy(
            zb_v.at[pl.ds(0, _ZPT % 16), :],
            acc.at[pl.ds(s * _ZPT + _ZPT - _ZPT % 16, _ZPT % 16), :],
        )


def _writeout(acc, out_hbm, c, s):
    pltpu.sync_copy(
        acc.at[pl.ds(s * _ZPT, _ZPT), :],
        out_hbm.at[c, pl.ds(s * _ZPT, _ZPT), :],
    )


@functools.cache
def _make_deg_kernel():
    return functools.partial(
        pl.kernel,
        out_type=jax.ShapeDtypeStruct((_NC, _ACC_ROWS, _D), jnp.float32),
        mesh=_sc_mesh(),
        scratch_types=[
            pltpu.VMEM((_CPW, _CHUNK), jnp.int32),
            pltpu.VMEM((_CHUNK, _D), jnp.float32),
            pltpu.VMEM((16, _D), jnp.float32),
            pltpu.VMEM_SHARED((_ACC_ROWS, _D), jnp.float32),
        ] + [pltpu.SemaphoreType.DMA] * _NB,
    )(_deg_body)


def _deg_body(dst_hbm, out_hbm, didx_v, ones_v, zb_v, acc, *sems):
    c = lax.axis_index("c")
    s = lax.axis_index("s")
    w = c * _NS + s

    pltpu.sync_copy(dst_hbm.at[pl.ds(w * _CPW, _CPW), :], didx_v)

    def ones_body(i, _):
        for j in range(_D // 16):
            ones_v[i, pl.ds(j * 16, 16)] = jnp.ones((16,), jnp.float32)
        return 0

    lax.fori_loop(0, _CHUNK, ones_body, 0)
    _zero_acc(zb_v, acc, s)
    plsc.subcore_barrier()

    for b in range(_NB):
        pltpu.async_copy(ones_v, acc.at[didx_v.at[b]], sems[b], add=True)

    def outer(t, _):
        for b in range(_NB):
            j = t * _NB + b
            pltpu.make_async_copy(ones_v, acc.at[didx_v.at[j]], sems[b]).wait()
            pltpu.async_copy(ones_v, acc.at[didx_v.at[j + _NB]], sems[b], add=True)
        return 0

    lax.fori_loop(0, _CPW // _NB - 1, outer, 0)
    for b in range(_NB):
        j = _CPW - _NB + b
        pltpu.make_async_copy(ones_v, acc.at[didx_v.at[j]], sems[b]).wait()
    plsc.subcore_barrier()
    _writeout(acc, out_hbm, c, s)


@functools.cache
def _make_agg_kernel():
    return functools.partial(
        pl.kernel,
        out_type=jax.ShapeDtypeStruct((_NC, _ACC_ROWS, _D), jnp.float32),
        mesh=_sc_mesh(),
        scratch_types=[
            pltpu.VMEM((_CHUNK,), jnp.int32),
            pltpu.VMEM((_CHUNK,), jnp.int32),
            pltpu.VMEM((_CHUNK,), jnp.int32),
            pltpu.VMEM((_CHUNK,), jnp.int32),
            pltpu.VMEM((_CHUNK,), jnp.int32),
            pltpu.VMEM((_CHUNK,), jnp.int32),
            pltpu.VMEM((_CHUNK, _D), jnp.float32),
            pltpu.VMEM((_CHUNK, _D), jnp.float32),
            pltpu.VMEM((_CHUNK, _D), jnp.float32),
            pltpu.VMEM_SHARED((_ACC_ROWS, _D), jnp.float32),
        ] + [pltpu.SemaphoreType.DMA] * 6,
    )(_agg_body)


def _agg_body(g0_hbm, g1_hbm, src_hbm, dst_hbm, out_hbm,
              sidx0, sidx1, sidx2, didx0, didx1, didx2,
              rows0, rows1, rows2, acc,
              si0, si1, si2, sg0, sg1, sg2):
    sidx = (sidx0, sidx1, sidx2)
    didx = (didx0, didx1, didx2)
    rows = (rows0, rows1, rows2)
    sem_i = (si0, si1, si2)
    sem_g = (sg0, sg1, sg2)

    c = lax.axis_index("c")
    s = lax.axis_index("s")

    def zrow_body(i, _):
        for jj in range(_D // 16):
            rows0[i, pl.ds(jj * 16, 16)] = jnp.zeros((16,), jnp.float32)
        return 0

    lax.fori_loop(0, _CHUNK, zrow_body, 0)
    for k in range(_ZPT // _CHUNK):
        pltpu.sync_copy(rows0,
                        acc.at[pl.ds(s * _ZPT + k * _CHUNK, _CHUNK), :])
    if _ZPT % _CHUNK:
        pltpu.sync_copy(
            rows0.at[pl.ds(0, _ZPT % _CHUNK), :],
            acc.at[pl.ds(s * _ZPT + _ZPT - _ZPT % _CHUNK, _ZPT % _CHUNK), :],
        )
    plsc.subcore_barrier()

    def pipeline(base0, cpw, g_hbm):
        def idx_dma(j, b):
            pltpu.async_copy(src_hbm.at[base0 + j], sidx[b], sem_i[b])
            pltpu.async_copy(dst_hbm.at[base0 + j], didx[b], sem_i[b])

        def idx_wait(j, b):
            pltpu.make_async_copy(src_hbm.at[base0 + j], sidx[b], sem_i[b]).wait()
            pltpu.make_async_copy(dst_hbm.at[base0 + j], didx[b], sem_i[b]).wait()

        def body(j, b):
            b2 = (b + 2) % 3

            @pl.when(j + 2 < cpw)
            def _():
                idx_wait(j + 2, b2)
                pltpu.async_copy(g_hbm.at[sidx[b2]], rows[b2], sem_g[b2])

            pltpu.make_async_copy(
                g_hbm.at[sidx[b]], rows[b], sem_g[b]).wait()
            pltpu.sync_copy(rows[b], acc.at[didx[b]], add=True)

            @pl.when(j + 3 < cpw)
            def _():
                idx_dma(j + 3, b)

        for b in range(3):
            idx_dma(b, b)
        for b in range(2):
            idx_wait(b, b)
            pltpu.async_copy(g_hbm.at[sidx[b]], rows[b], sem_g[b])

        def step(t, _):
            for b in range(3):
                body(3 * t + b, b)
            return 0

        lax.fori_loop(0, cpw // 3, step, 0)
        for j in range(cpw - cpw % 3, cpw):
            body(j, j % 3)

    if _CPW0:
        @pl.when(c == 0)
        def _():
            pipeline(s * _CPW0, _CPW0, g0_hbm)

    if _CPW1:
        @pl.when(c == 1)
        def _():
            pipeline(_NS * _CPW0 + s * _CPW1, _CPW1, g1_hbm)

    plsc.subcore_barrier()
    _writeout(acc, out_hbm, c, s)


_BN = 1000


def _t0_body(dp_ref, x_ref, w_ref, g_ref, g2_ref, dinv_ref):
    deg = dp_ref[0, :, 0:1] + dp_ref[1, :, 0:1] + 1.0
    dinv = lax.rsqrt(deg)
    h = jnp.dot(x_ref[...], w_ref[...], preferred_element_type=jnp.float32)
    g = dinv * h
    g_ref[...] = g
    g2_ref[...] = g
    dinv_ref[...] = dinv


def _t0(deg_partials, x, w0):
    return pl.pallas_call(
        _t0_body,
        grid=(_N // _BN,),
        in_specs=[
            pl.BlockSpec((_NC, _BN, _D), lambda i: (0, i, 0)),
            pl.BlockSpec((_BN, _D), lambda i: (i, 0)),
            pl.BlockSpec((_D, _D), lambda i: (0, 0)),
        ],
        out_specs=[
            pl.BlockSpec((_BN, _D), lambda i: (i, 0)),
            pl.BlockSpec((_BN, _D), lambda i: (i, 0)),
            pl.BlockSpec((_BN, 1), lambda i: (i, 0)),
        ],
        out_shape=[
            jax.ShapeDtypeStruct((_N, _D), jnp.float32),
            jax.ShapeDtypeStruct((_N, _D), jnp.float32),
            jax.ShapeDtypeStruct((_N, 1), jnp.float32),
        ],
    )(deg_partials, x, w0)


def _t12_body(p_ref, g_ref, dinv_ref, b_ref, w_ref, gn_ref, gn2_ref):
    dinv = dinv_ref[...]
    xn = dinv * (p_ref[0] + p_ref[1] + g_ref[...]) + b_ref[...]
    h = jnp.dot(xn, w_ref[...], preferred_element_type=jnp.float32)
    gn = dinv * h
    gn_ref[...] = gn
    gn2_ref[...] = gn


def _t12(partials, g, dinv, b, w):
    return pl.pallas_call(
        _t12_body,
        grid=(_N // _BN,),
        in_specs=[
            pl.BlockSpec((_NC, _BN, _D), lambda i: (0, i, 0)),
            pl.BlockSpec((_BN, _D), lambda i: (i, 0)),
            pl.BlockSpec((_BN, 1), lambda i: (i, 0)),
            pl.BlockSpec((1, _D), lambda i: (0, 0)),
            pl.BlockSpec((_D, _D), lambda i: (0, 0)),
        ],
        out_specs=[
            pl.BlockSpec((_BN, _D), lambda i: (i, 0)),
            pl.BlockSpec((_BN, _D), lambda i: (i, 0)),
        ],
        out_shape=[
            jax.ShapeDtypeStruct((_N, _D), jnp.float32),
            jax.ShapeDtypeStruct((_N, _D), jnp.float32),
        ],
    )(partials, g, dinv, b, w)


def _t3_body(p_ref, g_ref, dinv_ref, b_ref, wr_ref, br_ref, batch_ref,
             w_out_ref, cnt_ref):
    dinv = dinv_ref[...]
    x3 = dinv * (p_ref[0] + p_ref[1] + g_ref[...]) + b_ref[...]
    z = jnp.dot(x3, wr_ref[...], preferred_element_type=jnp.float32) + br_ref[...]
    gate = 1.0 / (1.0 + jnp.exp(-z))
    w_out_ref[...] = gate * x3

    @pl.when(pl.program_id(0) == 0)
    def _():
        cnt_ref[...] = jnp.zeros_like(cnt_ref)

    gids = lax.broadcasted_iota(jnp.int32, (1, _G), 1)
    eq = (batch_ref[...] == gids).astype(jnp.int32)
    cnt_ref[...] += jnp.sum(eq, axis=0, keepdims=True)


def _t3(partials, g, dinv, b, wr, br, batch2d):
    return pl.pallas_call(
        _t3_body,
        grid=(_N // _BN,),
        in_specs=[
            pl.BlockSpec((_NC, _BN, _D), lambda i: (0, i, 0)),
            pl.BlockSpec((_BN, _D), lambda i: (i, 0)),
            pl.BlockSpec((_BN, 1), lambda i: (i, 0)),
            pl.BlockSpec((1, _D), lambda i: (0, 0)),
            pl.BlockSpec((_D, 1), lambda i: (0, 0)),
            pl.BlockSpec((1, 1), lambda i: (0, 0)),
            pl.BlockSpec((_BN, 1), lambda i: (i, 0)),
        ],
        out_specs=[
            pl.BlockSpec((_BN, _D), lambda i: (i, 0)),
            pl.BlockSpec((1, _G), lambda i: (0, 0)),
        ],
        out_shape=[
            jax.ShapeDtypeStruct((_N, _D), jnp.float32),
            jax.ShapeDtypeStruct((1, _G), jnp.int32),
        ],
    )(partials, g, dinv, b, wr, br, batch2d)


_BR = 32
_NPAD = 10016


def _t4_body(w_ref, cnt_ref, out_ref):
    neg_inf = jnp.float32(-jnp.inf)

    def grp_body(grp, start):
        srows = []
        mrows = []
        for u in range(8):
            gi = grp * 8 + u
            cnt = cnt_ref[0, gi]
            end = start + cnt
            kb0 = start // _BR
            nblk = jnp.where(cnt > 0, (end - 1) // _BR - kb0 + 1, 0)

            def blk_body(t, carry):
                s_acc, m_acc = carry
                kb = kb0 + t
                blk = w_ref[pl.ds(kb * _BR, _BR), :]
                ridx = kb * _BR + lax.broadcasted_iota(jnp.int32, (_BR, 1), 0)
                msk = (ridx >= start) & (ridx < end)
                s_acc = s_acc + jnp.where(msk, blk, 0.0)
                m_acc = jnp.maximum(m_acc, jnp.where(msk, blk, neg_inf))
                return (s_acc, m_acc)

            s_acc, m_acc = lax.fori_loop(
                0, nblk, blk_body,
                (jnp.zeros((_BR, _D), jnp.float32),
                 jnp.full((_BR, _D), neg_inf, jnp.float32)),
            )
            srows.append(jnp.sum(s_acc, axis=0, keepdims=True))
            mrows.append(jnp.max(m_acc, axis=0, keepdims=True))
            start = end
        base = pl.multiple_of(grp * 8, 8)
        out_ref[pl.ds(base, 8), 0:_D] = jnp.concatenate(srows, axis=0)
        out_ref[pl.ds(base, 8), _D:2 * _D] = jnp.concatenate(mrows, axis=0)
        return start

    lax.fori_loop(0, _G // 8, grp_body, jnp.int32(0))


def _t4(weighted_pad, counts):
    return pl.pallas_call(
        _t4_body,
        in_specs=[
            pl.BlockSpec(memory_space=pltpu.VMEM),
            pl.BlockSpec(memory_space=pltpu.SMEM),
        ],
        out_specs=pl.BlockSpec(memory_space=pltpu.VMEM),
        out_shape=jax.ShapeDtypeStruct((_G, 2 * _D), jnp.float32),
    )(weighted_pad, counts)


@jax.jit
def kernel(x, edge_index, batch, W0, b0, W1, b1, W2, b2, Wr, br):
    src = edge_index[0]
    dst = edge_index[1]
    pad = _EPAD - _E
    src2 = jnp.concatenate([src, jnp.zeros((pad,), jnp.int32)]).reshape(_NCHUNK, _CHUNK)
    dst2 = jnp.concatenate([dst, jnp.full((pad,), _DUMMY, jnp.int32)]).reshape(_NCHUNK, _CHUNK)

    deg_partials = _make_deg_kernel()(dst2)

    g, g2, dinv = _t0(deg_partials, x, W0)
    p = _make_agg_kernel()(g, g2, src2, dst2)
    g, g2 = _t12(p, g, dinv, b0.reshape(1, _D), W1)
    p = _make_agg_kernel()(g, g2, src2, dst2)
    g, g2 = _t12(p, g, dinv, b1.reshape(1, _D), W2)
    p = _make_agg_kernel()(g, g2, src2, dst2)

    weighted, counts = _t3(p, g, dinv, b2.reshape(1, _D), Wr,
                           br.reshape(1, 1), batch.reshape(_N, 1))
    weighted_pad = jnp.pad(weighted, ((0, _NPAD - _N), (0, 0)))
    return _t4(weighted_pad, counts)


# speedup vs baseline: 1.1026x; 1.1026x over previous
import functools

import jax
import jax.numpy as jnp
from jax import lax
from jax.experimental import pallas as pl
from jax.experimental.pallas import tpu as pltpu
from jax.experimental.pallas import tpu_sc as plsc

_N = 10000
_D = 128
_G = 128
_E = 320000

_NC = 2
_NS = 16
_CHUNK = 128
_CPW = 80
_CPW0 = 80
_CPW1 = 80
_HP = 40
_NB = 2
_NCHUNK = _NC * _NS * _CPW
_EPAD = _NCHUNK * _CHUNK
_ACC_ROWS = 10240
_ZPT = _ACC_ROWS // _NS
_DUMMY = _N


def _sc_mesh():
    return plsc.VectorSubcoreMesh(
        core_axis_name="c", subcore_axis_name="s",
        num_cores=_NC, num_subcores=_NS)


def _zero_acc(zb_v, acc, s):
    for i in range(16):
        for j in range(_D // 16):
            zb_v[i, pl.ds(j * 16, 16)] = jnp.zeros((16,), jnp.float32)

    def zero_body(k, _):
        pltpu.sync_copy(zb_v, acc.at[pl.ds(s * _ZPT + k * 16, 16), :])
        return 0

    lax.fori_loop(0, _ZPT // 16, zero_body, 0)


def _writeout(acc, out_hbm, c, s):
    pltpu.sync_copy(
        acc.at[pl.ds(s * _ZPT, _ZPT), :],
        out_hbm.at[c, pl.ds(s * _ZPT, _ZPT), :],
    )


@functools.cache
def _make_deg_kernel():
    return functools.partial(
        pl.kernel,
        out_type=jax.ShapeDtypeStruct((_NC, _ACC_ROWS, _D), jnp.float32),
        mesh=_sc_mesh(),
        scratch_types=[
            pltpu.VMEM((_CPW, _CHUNK), jnp.int32),
            pltpu.VMEM((_CHUNK, _D), jnp.float32),
            pltpu.VMEM((16, _D), jnp.float32),
            pltpu.VMEM_SHARED((_ACC_ROWS, _D), jnp.float32),
        ] + [pltpu.SemaphoreType.DMA] * _NB,
    )(_deg_body)


def _deg_body(dst_hbm, out_hbm, didx_v, ones_v, zb_v, acc, *sems):
    c = lax.axis_index("c")
    s = lax.axis_index("s")
    w = c * _NS + s

    pltpu.sync_copy(dst_hbm.at[pl.ds(w * _CPW, _CPW), :], didx_v)

    def ones_body(i, _):
        for j in range(_D // 16):
            ones_v[i, pl.ds(j * 16, 16)] = jnp.ones((16,), jnp.float32)
        return 0

    lax.fori_loop(0, _CHUNK, ones_body, 0)
    _zero_acc(zb_v, acc, s)
    plsc.subcore_barrier()

    for b in range(_NB):
        pltpu.async_copy(ones_v, acc.at[didx_v.at[b]], sems[b], add=True)

    def outer(t, _):
        for b in range(_NB):
            j = t * _NB + b
            pltpu.make_async_copy(ones_v, acc.at[didx_v.at[j]], sems[b]).wait()
            pltpu.async_copy(ones_v, acc.at[didx_v.at[j + _NB]], sems[b], add=True)
        return 0

    lax.fori_loop(0, _CPW // _NB - 1, outer, 0)
    for b in range(_NB):
        j = _CPW - _NB + b
        pltpu.make_async_copy(ones_v, acc.at[didx_v.at[j]], sems[b]).wait()
    plsc.subcore_barrier()
    _writeout(acc, out_hbm, c, s)


@functools.cache
def _make_agg_kernel():
    return functools.partial(
        pl.kernel,
        out_type=jax.ShapeDtypeStruct((_NC, _ACC_ROWS, _D), jnp.float32),
        mesh=_sc_mesh(),
        scratch_types=[
            pltpu.VMEM((_CHUNK,), jnp.int32),
            pltpu.VMEM((_CHUNK,), jnp.int32),
            pltpu.VMEM((_CHUNK,), jnp.int32),
            pltpu.VMEM((_CHUNK,), jnp.int32),
            pltpu.VMEM((_CHUNK, _D), jnp.float32),
            pltpu.VMEM((_CHUNK, _D), jnp.float32),
            pltpu.VMEM((16, _D), jnp.float32),
            pltpu.VMEM_SHARED((_ACC_ROWS, _D), jnp.float32),
        ] + [pltpu.SemaphoreType.DMA] * 4,
    )(_agg_body)


def _agg_body(g0_hbm, g1_hbm, src_hbm, dst_hbm, out_hbm,
              sidx0, sidx1, didx0, didx1, rows0, rows1, zb_v, acc,
              si0, si1, sg0, sg1):
    sidx = (sidx0, sidx1)
    didx = (didx0, didx1)
    rows = (rows0, rows1)
    sem_i = (si0, si1)
    sem_g = (sg0, sg1)

    c = lax.axis_index("c")
    s = lax.axis_index("s")

    _zero_acc(zb_v, acc, s)
    plsc.subcore_barrier()

    def pipeline(base0, cpw, g_hbm):
        def idx_dma(j, b):
            pltpu.async_copy(src_hbm.at[base0 + j], sidx[b], sem_i[b])
            pltpu.async_copy(dst_hbm.at[base0 + j], didx[b], sem_i[b])

        def idx_wait(j, b):
            pltpu.make_async_copy(src_hbm.at[base0 + j], sidx[b], sem_i[b]).wait()
            pltpu.make_async_copy(dst_hbm.at[base0 + j], didx[b], sem_i[b]).wait()

        idx_dma(0, 0)
        idx_dma(1, 1)
        idx_wait(0, 0)
        pltpu.async_copy(g_hbm.at[sidx[0]], rows[0], sem_g[0])

        def step(t, _):
            for b in range(2):
                j = 2 * t + b
                bn = 1 - b

                @pl.when(j + 1 < cpw)
                def _():
                    idx_wait(j + 1, bn)
                    pltpu.async_copy(g_hbm.at[sidx[bn]], rows[bn], sem_g[bn])

                pltpu.make_async_copy(
                    g_hbm.at[sidx[b]], rows[b], sem_g[b]).wait()
                pltpu.sync_copy(rows[b], acc.at[didx[b]], add=True)

                @pl.when(j + 2 < cpw)
                def _():
                    idx_dma(j + 2, b)

            return 0

        lax.fori_loop(0, cpw // 2, step, 0)

    if _CPW0:
        @pl.when(c == 0)
        def _():
            pipeline(s * _CPW0, _CPW0, g0_hbm)

    if _CPW1:
        @pl.when(c == 1)
        def _():
            pipeline(_NS * _CPW0 + s * _CPW1, _CPW1, g1_hbm)

    plsc.subcore_barrier()
    _writeout(acc, out_hbm, c, s)


_BN = 1000


def _t0_body(dp_ref, x_ref, w_ref, g_ref, g2_ref, dinv_ref):
    deg = dp_ref[0, :, 0:1] + dp_ref[1, :, 0:1] + 1.0
    dinv = lax.rsqrt(deg)
    h = jnp.dot(x_ref[...], w_ref[...], preferred_element_type=jnp.float32)
    g = dinv * h
    g_ref[...] = g
    g2_ref[...] = g
    dinv_ref[...] = dinv


def _t0(deg_partials, x, w0):
    return pl.pallas_call(
        _t0_body,
        grid=(_N // _BN,),
        in_specs=[
            pl.BlockSpec((_NC, _BN, _D), lambda i: (0, i, 0)),
            pl.BlockSpec((_BN, _D), lambda i: (i, 0)),
            pl.BlockSpec((_D, _D), lambda i: (0, 0)),
        ],
        out_specs=[
            pl.BlockSpec((_BN, _D), lambda i: (i, 0)),
            pl.BlockSpec((_BN, _D), lambda i: (i, 0)),
            pl.BlockSpec((_BN, 1), lambda i: (i, 0)),
        ],
        out_shape=[
            jax.ShapeDtypeStruct((_N, _D), jnp.float32),
            jax.ShapeDtypeStruct((_N, _D), jnp.float32),
            jax.ShapeDtypeStruct((_N, 1), jnp.float32),
        ],
    )(deg_partials, x, w0)


def _t12_body(p_ref, g_ref, dinv_ref, b_ref, w_ref, gn_ref, gn2_ref):
    dinv = dinv_ref[...]
    xn = dinv * (p_ref[0] + p_ref[1] + g_ref[...]) + b_ref[...]
    h = jnp.dot(xn, w_ref[...], preferred_element_type=jnp.float32)
    gn = dinv * h
    gn_ref[...] = gn
    gn2_ref[...] = gn


def _t12(partials, g, dinv, b, w):
    return pl.pallas_call(
        _t12_body,
        grid=(_N // _BN,),
        in_specs=[
            pl.BlockSpec((_NC, _BN, _D), lambda i: (0, i, 0)),
            pl.BlockSpec((_BN, _D), lambda i: (i, 0)),
            pl.BlockSpec((_BN, 1), lambda i: (i, 0)),
            pl.BlockSpec((1, _D), lambda i: (0, 0)),
            pl.BlockSpec((_D, _D), lambda i: (0, 0)),
        ],
        out_specs=[
            pl.BlockSpec((_BN, _D), lambda i: (i, 0)),
            pl.BlockSpec((_BN, _D), lambda i: (i, 0)),
        ],
        out_shape=[
            jax.ShapeDtypeStruct((_N, _D), jnp.float32),
            jax.ShapeDtypeStruct((_N, _D), jnp.float32),
        ],
    )(partials, g, dinv, b, w)


def _t3_body(p_ref, g_ref, dinv_ref, b_ref, wr_ref, br_ref, batch_ref,
             w_out_ref, cnt_ref):
    dinv = dinv_ref[...]
    x3 = dinv * (p_ref[0] + p_ref[1] + g_ref[...]) + b_ref[...]
    z = jnp.dot(x3, wr_ref[...], preferred_element_type=jnp.float32) + br_ref[...]
    gate = 1.0 / (1.0 + jnp.exp(-z))
    w_out_ref[...] = gate * x3

    @pl.when(pl.program_id(0) == 0)
    def _():
        cnt_ref[...] = jnp.zeros_like(cnt_ref)

    gids = lax.broadcasted_iota(jnp.int32, (1, _G), 1)
    eq = (batch_ref[...] == gids).astype(jnp.int32)
    cnt_ref[...] += jnp.sum(eq, axis=0, keepdims=True)


def _t3(partials, g, dinv, b, wr, br, batch2d):
    return pl.pallas_call(
        _t3_body,
        grid=(_N // _BN,),
        in_specs=[
            pl.BlockSpec((_NC, _BN, _D), lambda i: (0, i, 0)),
            pl.BlockSpec((_BN, _D), lambda i: (i, 0)),
            pl.BlockSpec((_BN, 1), lambda i: (i, 0)),
            pl.BlockSpec((1, _D), lambda i: (0, 0)),
            pl.BlockSpec((_D, 1), lambda i: (0, 0)),
            pl.BlockSpec((1, 1), lambda i: (0, 0)),
            pl.BlockSpec((_BN, 1), lambda i: (i, 0)),
        ],
        out_specs=[
            pl.BlockSpec((_BN, _D), lambda i: (i, 0)),
            pl.BlockSpec((1, _G), lambda i: (0, 0)),
        ],
        out_shape=[
            jax.ShapeDtypeStruct((_N, _D), jnp.float32),
            jax.ShapeDtypeStruct((1, _G), jnp.int32),
        ],
    )(partials, g, dinv, b, wr, br, batch2d)


_BR = 32
_NPAD = 10016


def _t4_body(w_ref, cnt_ref, out_ref):
    neg_inf = jnp.float32(-jnp.inf)

    def grp_body(grp, start):
        srows = []
        mrows = []
        for u in range(8):
            gi = grp * 8 + u
            cnt = cnt_ref[0, gi]
            end = start + cnt
            kb0 = start // _BR
            nblk = jnp.where(cnt > 0, (end - 1) // _BR - kb0 + 1, 0)

            def blk_body(t, carry):
                s_acc, m_acc = carry
                kb = kb0 + t
                blk = w_ref[pl.ds(kb * _BR, _BR), :]
                ridx = kb * _BR + lax.broadcasted_iota(jnp.int32, (_BR, 1), 0)
                msk = (ridx >= start) & (ridx < end)
                s_acc = s_acc + jnp.where(msk, blk, 0.0)
                m_acc = jnp.maximum(m_acc, jnp.where(msk, blk, neg_inf))
                return (s_acc, m_acc)

            s_acc, m_acc = lax.fori_loop(
                0, nblk, blk_body,
                (jnp.zeros((_BR, _D), jnp.float32),
                 jnp.full((_BR, _D), neg_inf, jnp.float32)),
            )
            srows.append(jnp.sum(s_acc, axis=0, keepdims=True))
            mrows.append(jnp.max(m_acc, axis=0, keepdims=True))
            start = end
        base = pl.multiple_of(grp * 8, 8)
        out_ref[pl.ds(base, 8), 0:_D] = jnp.concatenate(srows, axis=0)
        out_ref[pl.ds(base, 8), _D:2 * _D] = jnp.concatenate(mrows, axis=0)
        return start

    lax.fori_loop(0, _G // 8, grp_body, jnp.int32(0))


def _t4(weighted_pad, counts):
    return pl.pallas_call(
        _t4_body,
        in_specs=[
            pl.BlockSpec(memory_space=pltpu.VMEM),
            pl.BlockSpec(memory_space=pltpu.SMEM),
        ],
        out_specs=pl.BlockSpec(memory_space=pltpu.VMEM),
        out_shape=jax.ShapeDtypeStruct((_G, 2 * _D), jnp.float32),
    )(weighted_pad, counts)


@jax.jit
def kernel(x, edge_index, batch, W0, b0, W1, b1, W2, b2, Wr, br):
    src = edge_index[0]
    dst = edge_index[1]
    pad = _EPAD - _E
    src2 = jnp.concatenate([src, jnp.zeros((pad,), jnp.int32)]).reshape(_NCHUNK, _CHUNK)
    dst2 = jnp.concatenate([dst, jnp.full((pad,), _DUMMY, jnp.int32)]).reshape(_NCHUNK, _CHUNK)

    deg_partials = _make_deg_kernel()(dst2)

    g, g2, dinv = _t0(deg_partials, x, W0)
    p = _make_agg_kernel()(g, g2, src2, dst2)
    g, g2 = _t12(p, g, dinv, b0.reshape(1, _D), W1)
    p = _make_agg_kernel()(g, g2, src2, dst2)
    g, g2 = _t12(p, g, dinv, b1.reshape(1, _D), W2)
    p = _make_agg_kernel()(g, g2, src2, dst2)

    weighted, counts = _t3(p, g, dinv, b2.reshape(1, _D), Wr,
                           br.reshape(1, 1), batch.reshape(_N, 1))
    weighted_pad = jnp.pad(weighted, ((0, _NPAD - _N), (0, 0)))
    return _t4(weighted_pad, counts)
